# baseline (speedup 1.0000x reference)
import functools

import jax
import jax.numpy as jnp
from jax.experimental import pallas as pl
from jax.experimental.pallas import tpu as pltpu

S_BLK = 2048


def _ln_body(tab_ref, w_ref, b_ref, out_ref, scratch_ref):
    @pl.when(pl.program_id(1) == 0)
    def _compute():
        rows = tab_ref[...]
        mu = jnp.mean(rows, axis=1, keepdims=True)
        var = jnp.mean(rows * rows, axis=1, keepdims=True) - mu * mu
        normed = (rows - mu) * jax.lax.rsqrt(var + 1e-5)
        normed = normed * w_ref[...] + b_ref[...]
        scratch_ref[...] = normed.T

    out_ref[0] = scratch_ref[...]


@functools.partial(jax.jit, static_argnames=("seq_len", "batch"))
def _pos_embed(pos_table, ln_weight, ln_bias, seq_len, batch):
    d = pos_table.shape[1]
    grid = (seq_len // S_BLK, batch)
    return pl.pallas_call(
        _ln_body,
        grid=grid,
        in_specs=[
            pl.BlockSpec((S_BLK, d), lambda j, b: (j, 0)),
            pl.BlockSpec((1, d), lambda j, b: (0, 0)),
            pl.BlockSpec((1, d), lambda j, b: (0, 0)),
        ],
        out_specs=pl.BlockSpec((1, d, S_BLK), lambda j, b: (b, 0, j)),
        out_shape=jax.ShapeDtypeStruct((batch, d, seq_len), pos_table.dtype),
        scratch_shapes=[pltpu.VMEM((d, S_BLK), pos_table.dtype)],
    )(pos_table, ln_weight.reshape(1, d), ln_bias.reshape(1, d))


def kernel(x, pos_table, ln_weight, ln_bias):
    batch, _, seq_len = x.shape
    return _pos_embed(pos_table, ln_weight, ln_bias, seq_len, batch)
